# baseline (speedup 1.0000x reference)
import functools

import jax
import jax.numpy as jnp
from jax import lax
from jax.experimental import pallas as pl
from jax.experimental.pallas import tpu as pltpu
from jax.experimental.pallas import tpu_sc as plsc

VOCAB = 100000
HIDDEN = 256
EMBED = 128
BATCH = 1024
CTX = 50
CTX_PAD = 56

NUM_WORKERS = 32
ROWS_PER_WORKER = BATCH // NUM_WORKERS
LANES = 16
COL_CHUNKS = EMBED // LANES

VTILE = 4096
NT = -(-VOCAB // VTILE)


def _sc_pool_body(idx_hbm, table_hbm, out_hbm, idx_v, rows0, rows1, acc_v,
                  sem0, sem1):
    wid = lax.axis_index("s") * 2 + lax.axis_index("c")
    row_base = wid * ROWS_PER_WORKER
    idx_base = row_base * CTX_PAD

    pltpu.sync_copy(idx_hbm.at[pl.ds(idx_base, ROWS_PER_WORKER * CTX_PAD)],
                    idx_v)

    bufs = (rows0, rows1)
    sems = (sem0, sem1)

    def fire(b, buf, sem):
        return pltpu.async_copy(
            table_hbm.at[idx_v.at[pl.ds(b * CTX_PAD, CTX_PAD)]], buf, sem)

    cp0 = fire(0, bufs[0], sems[0])
    for b in range(ROWS_PER_WORKER):
        cur = bufs[b % 2]
        if b == 0:
            cp0.wait()
        if b + 1 < ROWS_PER_WORKER:
            nxt = fire(b + 1, bufs[(b + 1) % 2], sems[(b + 1) % 2])
        for c in range(COL_CHUNKS):
            def rbody(r, acc):
                return acc + cur[r, pl.ds(c * LANES, LANES)]
            acc = lax.fori_loop(1, CTX, rbody, cur[0, pl.ds(c * LANES, LANES)])
            acc_v[b, pl.ds(c * LANES, LANES)] = acc * (1.0 / CTX)
        if b + 1 < ROWS_PER_WORKER:
            nxt.wait()

    pltpu.sync_copy(acc_v, out_hbm.at[pl.ds(row_base, ROWS_PER_WORKER)])


@functools.cache
def _get_sc_pool():
    return pl.kernel(
        _sc_pool_body,
        out_type=jax.ShapeDtypeStruct((BATCH, EMBED), jnp.float32),
        mesh=plsc.VectorSubcoreMesh(core_axis_name="c", subcore_axis_name="s"),
        scratch_types=[
            pltpu.VMEM((ROWS_PER_WORKER * CTX_PAD,), jnp.int32),
            pltpu.VMEM((CTX_PAD, EMBED), jnp.float32),
            pltpu.VMEM((CTX_PAD, EMBED), jnp.float32),
            pltpu.VMEM((ROWS_PER_WORKER, EMBED), jnp.float32),
            pltpu.SemaphoreType.DMA,
            pltpu.SemaphoreType.DMA,
        ],
    )


def _tc_body(pooled_ref, wh_ref, bh_ref, wo_ref, bo_ref, out_ref,
             hidden_ref, s_ref):
    p = pl.program_id(0)
    j = pl.program_id(1)

    @pl.when((p == 0) & (j == 0))
    def _init():
        h = jnp.dot(pooled_ref[...], wh_ref[...],
                    preferred_element_type=jnp.float32) + bh_ref[...]
        hidden_ref[...] = jnp.maximum(h, 0.0).astype(jnp.bfloat16)
        s_ref[...] = jnp.zeros_like(s_ref)

    wo = wo_ref[...].astype(jnp.bfloat16)
    logits = jnp.dot(hidden_ref[...], wo,
                     preferred_element_type=jnp.float32) + bo_ref[...]

    @pl.when(p == 0)
    def _accum():
        col = j * VTILE + lax.broadcasted_iota(jnp.int32, (BATCH, VTILE), 1)
        ex = jnp.where(col < VOCAB, jnp.exp(logits), 0.0)
        s_ref[...] += jnp.sum(ex, axis=1, keepdims=True)

    @pl.when(p == 1)
    def _write():
        out_ref[...] = logits - jnp.log(s_ref[...])


def _tc_mlp(pooled, w_h, b_h, w_o, b_o):
    return pl.pallas_call(
        _tc_body,
        grid=(2, NT),
        in_specs=[
            pl.BlockSpec((BATCH, EMBED), lambda p, j: (0, 0)),
            pl.BlockSpec((EMBED, HIDDEN), lambda p, j: (0, 0)),
            pl.BlockSpec((1, HIDDEN), lambda p, j: (0, 0)),
            pl.BlockSpec((HIDDEN, VTILE), lambda p, j: (0, j)),
            pl.BlockSpec((1, VTILE), lambda p, j: (0, j)),
        ],
        out_specs=pl.BlockSpec((BATCH, VTILE),
                               lambda p, j: (0, jnp.where(p == 0, 0, j))),
        out_shape=jax.ShapeDtypeStruct((BATCH, VOCAB), jnp.float32),
        scratch_shapes=[
            pltpu.VMEM((BATCH, HIDDEN), jnp.bfloat16),
            pltpu.VMEM((BATCH, 1), jnp.float32),
        ],
    )(pooled, w_h, b_h, w_o, b_o)


def kernel(x, emb_table, W_h, b_h, W_o, b_o):
    xp = jnp.pad(x.astype(jnp.int32), ((0, 0), (0, CTX_PAD - CTX)))
    idx_flat = xp.reshape(-1)
    pooled = _get_sc_pool()(idx_flat, emb_table)
    return _tc_mlp(pooled, W_h, b_h.reshape(1, HIDDEN), W_o,
                   b_o.reshape(1, VOCAB))
